# baseline (speedup 1.0000x reference)
import jax
import jax.numpy as jnp
from jax import lax
from jax.experimental import pallas as pl
from jax.experimental.pallas import tpu as pltpu
from jax.experimental.pallas import tpu_sc as plsc

_B, _V, _N, _D = 1, 4, 10000, 128
_H, _F = 4, 32
_NEG = 0.2
_BV = _B * _V
_NP = 10112
_RPT = _NP // 16
_E2 = 160000 + _N
_C = 32
_CPT = 336
_EPT = _C * _CPT
_EP = _EPT * 16
_ACC_W = 144


def _proj_body(x_ref, w_ref, m_ref, h_ref, s_ref):
    h = jnp.dot(x_ref[...], w_ref[...], preferred_element_type=jnp.float32)
    h_ref[...] = h
    lr = jnp.where(h > 0, h, _NEG * h)
    s_ref[...] = jnp.dot(lr, m_ref[...], preferred_element_type=jnp.float32)


def _mlp_body(a_ref, r_ref, w1_ref, b1_ref, w2_ref, b2_ref, y_ref):
    blk = a_ref[...]
    inv = 1.0 / (blk[:, 128:144] + 1e-16)
    factor = jnp.dot(inv, r_ref[...], preferred_element_type=jnp.float32)
    un = blk[:, 0:128] * factor
    z = jnp.maximum(
        jnp.dot(un, w1_ref[...], preferred_element_type=jnp.float32) + b1_ref[...],
        0.0)
    y_ref[...] = (jnp.dot(z, w2_ref[...], preferred_element_type=jnp.float32)
                  + b2_ref[...])


def _sc_body(tab_ref, src3_ref, dst3_ref, out_ref,
             vsrc, vdst, sadj0, sadj1, dsts0, dsts1,
             rows0, rows1, scaled0, scaled1,
             acc, ga0, ga1, sc0, sc1):
    cid = lax.axis_index("c")
    sid = lax.axis_index("s")
    r0 = sid * _RPT
    zero16 = jnp.zeros((16,), jnp.float32)
    sadj = (sadj0, sadj1)
    dsts = (dsts0, dsts1)
    rows = (rows0, rows1)
    scaled = (scaled0, scaled1)
    gsa = (ga0, ga1)
    ssem = (sc0, sc1)

    ebase = sid * _EPT

    seg = _EPT // 8
    def iload(g, carry):
        off = g * seg
        pltpu.sync_copy(src3_ref.at[pl.ds(ebase + off, seg)],
                        vsrc.at[pl.ds(off, seg)])
        pltpu.sync_copy(dst3_ref.at[pl.ds(ebase + off, seg)],
                        vdst.at[pl.ds(off, seg)])
        return carry
    lax.fori_loop(0, 8, iload, 0)

    def waitg(s):
        pltpu.make_async_copy(tab_ref.at[sadj[s]], rows[s], gsa[s]).wait()

    def start_scatter(s):
        pltpu.async_copy(scaled[s], acc.at[dsts[s]], ssem[s], add=True)

    def wait_scatter(s):
        pltpu.make_async_copy(scaled[s], acc.at[dsts[s]], ssem[s]).wait()

    def compute(s):
        def edge(e, ecarry):
            exv = jnp.exp(rows[s][e, pl.ds(128, 16)])
            scaled[s][e, pl.ds(128, 16)] = exv
            for hh in range(_H):
                f = exv[hh]
                for j in (2 * hh, 2 * hh + 1):
                    sl = pl.ds(j * 16, 16)
                    scaled[s][e, sl] = rows[s][e, sl] * f
            return ecarry
        lax.fori_loop(0, _C, edge, 0, unroll=2)

    for vi in range(2):
        v = 2 * cid + vi
        voff = v * _NP

        def fill_gather(kk, s):
            base = kk * _C
            for i in range(_C // 16):
                sl = pl.ds(i * 16, 16)
                sadj[s][sl] = vsrc[pl.ds(base + i * 16, 16)] + voff
            pltpu.async_copy(tab_ref.at[sadj[s]], rows[s], gsa[s])

        def fill_dsts(kk, s):
            base = kk * _C
            for i in range(_C // 16):
                sl = pl.ds(i * 16, 16)
                dsts[s][sl] = vdst[pl.ds(base + i * 16, 16)]

        def zrow(r, carry):
            for j in range(_ACC_W // 16):
                scaled0[r, pl.ds(j * 16, 16)] = zero16
            return carry
        lax.fori_loop(0, _C, zrow, 0)
        off = 0
        while off < _RPT:
            cnt = min(_C, _RPT - off)
            pltpu.sync_copy(scaled0.at[pl.ds(0, cnt), :],
                            acc.at[pl.ds(r0 + off, cnt), :])
            off += cnt
        plsc.subcore_barrier()

        pltpu.async_copy(scaled0, out_ref.at[pl.ds(voff + r0, _C), :], sc0)
        pltpu.async_copy(scaled1, out_ref.at[pl.ds(voff + r0, _C), :], sc1)
        fill_gather(0, 0)
        fill_gather(1, 1)

        def pair(k2, carry):
            for s in (0, 1):
                kk = 2 * k2 + s
                waitg(s)
                wait_scatter(s)
                fill_dsts(kk, s)
                compute(s)
                start_scatter(s)
                fill_gather(jnp.minimum(kk + 2, _CPT - 1), s)
            return carry
        lax.fori_loop(0, _CPT // 2, pair, 0)
        waitg(0)
        waitg(1)
        wait_scatter(0)
        wait_scatter(1)
        plsc.subcore_barrier()

        off = 0
        while off < _RPT:
            cnt = min(128, _RPT - off)
            rsl = pl.ds(r0 + off, cnt)
            osl = pl.ds(voff + r0 + off, cnt)
            pltpu.sync_copy(acc.at[rsl, :], out_ref.at[osl, :])
            off += cnt


def _make_sc_call():
    mesh = plsc.VectorSubcoreMesh(core_axis_name="c", subcore_axis_name="s",
                                  num_cores=2, num_subcores=16)
    return pl.kernel(
        _sc_body,
        out_type=jax.ShapeDtypeStruct((_BV * _NP, _ACC_W), jnp.float32),
        mesh=mesh,
        scratch_types=(
            [
                pltpu.VMEM((_EPT,), jnp.int32),
                pltpu.VMEM((_EPT,), jnp.int32),
            ]
            + [pltpu.VMEM((_C,), jnp.int32) for _ in range(4)]
            + [
                pltpu.VMEM((_C, _ACC_W), jnp.float32),
                pltpu.VMEM((_C, _ACC_W), jnp.float32),
                pltpu.VMEM((_C, _ACC_W), jnp.float32),
                pltpu.VMEM((_C, _ACC_W), jnp.float32),
                pltpu.VMEM_SHARED((_NP, _ACC_W), jnp.float32),
            ]
            + [pltpu.SemaphoreType.DMA for _ in range(4)]
        ),
        compiler_params=pltpu.CompilerParams(use_tc_tiling_on_sc=False),
    )


def kernel(x, edge_index, W, att, W1, b1, W2, b2, bias):
    f32 = jnp.float32
    xf = x.reshape(_BV * _N, _D)

    att2 = att[0]
    oneh = jnp.repeat(jnp.eye(_H, dtype=f32), _F, axis=0)
    msrc = oneh * att2[:, :_F].reshape(-1)[:, None]
    mcat = jnp.concatenate([msrc, jnp.zeros((_H * _F, 12), f32)], axis=1)

    h_flat, scores = pl.pallas_call(
        _proj_body,
        grid=(125,),
        in_specs=[
            pl.BlockSpec((320, 128), lambda i: (i, 0)),
            pl.BlockSpec((128, 128), lambda i: (0, 0)),
            pl.BlockSpec((128, 16), lambda i: (0, 0)),
        ],
        out_specs=[
            pl.BlockSpec((320, 128), lambda i: (i, 0)),
            pl.BlockSpec((320, 16), lambda i: (i, 0)),
        ],
        out_shape=[
            jax.ShapeDtypeStruct((_BV * _N, 128), f32),
            jax.ShapeDtypeStruct((_BV * _N, 16), f32),
        ],
    )(xf, W, mcat)

    sc3 = scores.reshape(_BV, _N, 16)
    asrc = sc3[..., 0:4]
    c_s = jnp.max(asrc, axis=(1, 2), keepdims=True)
    pad_n = _NP - _N
    body = jnp.concatenate(
        [h_flat.reshape(_BV, _N, 128), asrc - c_s,
         jnp.full((_BV, _N, 12), -1e30, f32)], axis=-1)
    pad = jnp.concatenate(
        [jnp.zeros((_BV, pad_n, 128), f32),
         jnp.full((_BV, pad_n, 16), -1e30, f32)], axis=-1)
    tab = jnp.concatenate([body, pad], axis=1).reshape(_BV * _NP, _ACC_W)

    sl = jnp.arange(_N, dtype=edge_index.dtype)
    ei = jnp.concatenate(
        [edge_index, jnp.stack([sl, sl], axis=0),
         jnp.full((2, _EP - _E2), _N, edge_index.dtype)], axis=1)
    src3 = ei[0]
    dst3 = ei[1]

    agg = _make_sc_call()(tab, src3, dst3)

    r16 = jnp.concatenate(
        [jnp.repeat(jnp.eye(_H, dtype=f32), _F, axis=1),
         jnp.zeros((12, _H * _F), f32)], axis=0)
    b1r = b1.reshape(1, -1)
    b2b = (b2 + bias).reshape(1, -1)

    y = pl.pallas_call(
        _mlp_body,
        grid=(158,),
        in_specs=[
            pl.BlockSpec((256, _ACC_W), lambda i: (i, 0)),
            pl.BlockSpec((16, 128), lambda i: (0, 0)),
            pl.BlockSpec((128, 256), lambda i: (0, 0)),
            pl.BlockSpec((1, 256), lambda i: (0, 0)),
            pl.BlockSpec((256, 128), lambda i: (0, 0)),
            pl.BlockSpec((1, 128), lambda i: (0, 0)),
        ],
        out_specs=pl.BlockSpec((256, 128), lambda i: (i, 0)),
        out_shape=jax.ShapeDtypeStruct((_BV * _NP, 128), f32),
    )(agg, r16, W1, b1r, W2, b2b)

    y4 = y.reshape(_BV, _NP, 128)[:, :_N, :]
    return y4.reshape(_B, _V, _N, _H * _F)


# speedup vs baseline: 1.1569x; 1.1569x over previous
import jax
import jax.numpy as jnp
from jax import lax
from jax.experimental import pallas as pl
from jax.experimental.pallas import tpu as pltpu
from jax.experimental.pallas import tpu_sc as plsc

_B, _V, _N, _D = 1, 4, 10000, 128
_H, _F = 4, 32
_NEG = 0.2
_BV = _B * _V
_NP = 10112
_RPT = _NP // 16
_E2 = 160000 + _N
_C = 32
_CPT = 336
_EPT = _C * _CPT
_EP = _EPT * 16
_ACC_W = 144


def _proj_body(x_ref, w_ref, m_ref, h_ref, s_ref):
    h = jnp.dot(x_ref[...], w_ref[...], preferred_element_type=jnp.float32)
    h_ref[...] = h
    lr = jnp.where(h > 0, h, _NEG * h)
    s_ref[...] = jnp.dot(lr, m_ref[...], preferred_element_type=jnp.float32)


def _mlp_body(a_ref, r_ref, w1_ref, b1_ref, w2_ref, b2_ref, y_ref):
    blk = a_ref[...]
    inv = 1.0 / (blk[:, 128:144] + 1e-16)
    factor = jnp.dot(inv, r_ref[...], preferred_element_type=jnp.float32)
    un = blk[:, 0:128] * factor
    z = jnp.maximum(
        jnp.dot(un, w1_ref[...], preferred_element_type=jnp.float32) + b1_ref[...],
        0.0)
    y_ref[...] = (jnp.dot(z, w2_ref[...], preferred_element_type=jnp.float32)
                  + b2_ref[...])


def _sc_body(ts_ref, hb_ref, src3_ref, dst3_ref, out_ref,
             vsrc, vdst, sadj0, sadj1, dsts0, dsts1,
             srows0, srows1, rows0, rows1, scaled0, scaled1,
             acc, ga0, ga1, gc0, gc1, sc0, sc1):
    cid = lax.axis_index("c")
    sid = lax.axis_index("s")
    r0 = sid * _RPT
    zero16 = jnp.zeros((16,), jnp.float32)
    sadj = (sadj0, sadj1)
    dsts = (dsts0, dsts1)
    srows = (srows0, srows1)
    rows = (rows0, rows1)
    scaled = (scaled0, scaled1)
    gsa = (ga0, ga1)
    gsc = (gc0, gc1)
    ssem = (sc0, sc1)

    ebase = sid * _EPT

    seg = _EPT // 8
    def iload(g, carry):
        off = g * seg
        pltpu.sync_copy(src3_ref.at[pl.ds(ebase + off, seg)],
                        vsrc.at[pl.ds(off, seg)])
        pltpu.sync_copy(dst3_ref.at[pl.ds(ebase + off, seg)],
                        vdst.at[pl.ds(off, seg)])
        return carry
    lax.fori_loop(0, 8, iload, 0)

    def waitg(s):
        pltpu.make_async_copy(ts_ref.at[sadj[s]], srows[s], gsa[s]).wait()
        pltpu.make_async_copy(hb_ref.at[sadj[s]], rows[s], gsc[s]).wait()

    def start_scatter(s):
        pltpu.async_copy(scaled[s], acc.at[dsts[s]], ssem[s], add=True)

    def wait_scatter(s):
        pltpu.make_async_copy(scaled[s], acc.at[dsts[s]], ssem[s]).wait()

    def compute(s):
        msk = jnp.int32(-65536)
        def edge(e, ecarry):
            exv = jnp.exp(srows[s][e, :])
            scaled[s][e, pl.ds(128, 16)] = exv
            for hh in range(_H):
                f = exv[hh]
                w = rows[s][e, pl.ds(16 * hh, 16)]
                a = lax.bitcast_convert_type(w << 16, jnp.float32)
                b = lax.bitcast_convert_type(w & msk, jnp.float32)
                scaled[s][e, pl.ds(32 * hh, 16)] = a * f
                scaled[s][e, pl.ds(32 * hh + 16, 16)] = b * f
            return ecarry
        lax.fori_loop(0, _C, edge, 0, unroll=2)

    for vi in range(2):
        v = 2 * cid + vi
        voff = v * _NP

        def fill_gather(kk, s):
            base = kk * _C
            for i in range(_C // 16):
                sl = pl.ds(i * 16, 16)
                sadj[s][sl] = vsrc[pl.ds(base + i * 16, 16)] + voff
            pltpu.async_copy(ts_ref.at[sadj[s]], srows[s], gsa[s])
            pltpu.async_copy(hb_ref.at[sadj[s]], rows[s], gsc[s])

        def fill_dsts(kk, s):
            base = kk * _C
            for i in range(_C // 16):
                sl = pl.ds(i * 16, 16)
                dsts[s][sl] = vdst[pl.ds(base + i * 16, 16)]

        def zrow(r, carry):
            for j in range(_ACC_W // 16):
                scaled0[r, pl.ds(j * 16, 16)] = zero16
            return carry
        lax.fori_loop(0, _C, zrow, 0)
        off = 0
        while off < _RPT:
            cnt = min(_C, _RPT - off)
            pltpu.sync_copy(scaled0.at[pl.ds(0, cnt), :],
                            acc.at[pl.ds(r0 + off, cnt), :])
            off += cnt
        plsc.subcore_barrier()

        pltpu.async_copy(scaled0, out_ref.at[pl.ds(voff + r0, _C), :], sc0)
        pltpu.async_copy(scaled1, out_ref.at[pl.ds(voff + r0, _C), :], sc1)
        fill_gather(0, 0)
        fill_gather(1, 1)

        def pair(k2, carry):
            for s in (0, 1):
                kk = 2 * k2 + s
                waitg(s)
                wait_scatter(s)
                fill_dsts(kk, s)
                compute(s)
                start_scatter(s)
                fill_gather(jnp.minimum(kk + 2, _CPT - 1), s)
            return carry
        lax.fori_loop(0, _CPT // 2, pair, 0)
        waitg(0)
        waitg(1)
        wait_scatter(0)
        wait_scatter(1)
        plsc.subcore_barrier()

        off = 0
        while off < _RPT:
            cnt = min(128, _RPT - off)
            rsl = pl.ds(r0 + off, cnt)
            osl = pl.ds(voff + r0 + off, cnt)
            pltpu.sync_copy(acc.at[rsl, :], out_ref.at[osl, :])
            off += cnt


def _make_sc_call():
    mesh = plsc.VectorSubcoreMesh(core_axis_name="c", subcore_axis_name="s",
                                  num_cores=2, num_subcores=16)
    return pl.kernel(
        _sc_body,
        out_type=jax.ShapeDtypeStruct((_BV * _NP, _ACC_W), jnp.float32),
        mesh=mesh,
        scratch_types=(
            [
                pltpu.VMEM((_EPT,), jnp.int32),
                pltpu.VMEM((_EPT,), jnp.int32),
            ]
            + [pltpu.VMEM((_C,), jnp.int32) for _ in range(4)]
            + [
                pltpu.VMEM((_C, 16), jnp.float32),
                pltpu.VMEM((_C, 16), jnp.float32),
                pltpu.VMEM((_C, 64), jnp.int32),
                pltpu.VMEM((_C, 64), jnp.int32),
                pltpu.VMEM((_C, _ACC_W), jnp.float32),
                pltpu.VMEM((_C, _ACC_W), jnp.float32),
                pltpu.VMEM_SHARED((_NP, _ACC_W), jnp.float32),
            ]
            + [pltpu.SemaphoreType.DMA for _ in range(6)]
        ),
        compiler_params=pltpu.CompilerParams(use_tc_tiling_on_sc=False),
    )


def kernel(x, edge_index, W, att, W1, b1, W2, b2, bias):
    f32 = jnp.float32
    xf = x.reshape(_BV * _N, _D)

    att2 = att[0]
    oneh = jnp.repeat(jnp.eye(_H, dtype=f32), _F, axis=0)
    msrc = oneh * att2[:, :_F].reshape(-1)[:, None]
    mcat = jnp.concatenate([msrc, jnp.zeros((_H * _F, 12), f32)], axis=1)

    h_flat, scores = pl.pallas_call(
        _proj_body,
        grid=(125,),
        in_specs=[
            pl.BlockSpec((320, 128), lambda i: (i, 0)),
            pl.BlockSpec((128, 128), lambda i: (0, 0)),
            pl.BlockSpec((128, 16), lambda i: (0, 0)),
        ],
        out_specs=[
            pl.BlockSpec((320, 128), lambda i: (i, 0)),
            pl.BlockSpec((320, 16), lambda i: (i, 0)),
        ],
        out_shape=[
            jax.ShapeDtypeStruct((_BV * _N, 128), f32),
            jax.ShapeDtypeStruct((_BV * _N, 16), f32),
        ],
    )(xf, W, mcat)

    sc3 = scores.reshape(_BV, _N, 16)
    asrc = sc3[..., 0:4]
    c_s = jnp.max(asrc, axis=(1, 2), keepdims=True)
    pad_n = _NP - _N
    zpad12 = jnp.zeros((_BV, _N, 12), f32)
    neg_row = jnp.full((_BV, pad_n, 16), -1e30, f32)
    ts = jnp.concatenate(
        [jnp.concatenate([asrc - c_s, zpad12], axis=-1), neg_row], axis=1)
    ts = ts.reshape(_BV * _NP, 16)
    hb = jnp.concatenate(
        [h_flat.reshape(_BV, _N, 128), jnp.zeros((_BV, pad_n, 128), f32)],
        axis=1).reshape(_BV * _NP, 64, 2).astype(jnp.bfloat16)
    hb = lax.bitcast_convert_type(hb, jnp.int32)

    sl = jnp.arange(_N, dtype=edge_index.dtype)
    ei = jnp.concatenate(
        [edge_index, jnp.stack([sl, sl], axis=0),
         jnp.full((2, _EP - _E2), _N, edge_index.dtype)], axis=1)
    src3 = ei[0]
    dst3 = ei[1]

    agg = _make_sc_call()(ts, hb, src3, dst3)

    perm = []
    for hh in range(_H):
        perm += [32 * hh + 2 * t for t in range(16)]
        perm += [32 * hh + 2 * t + 1 for t in range(16)]
    w1p = W1[jnp.array(perm), :]
    r16 = jnp.concatenate(
        [jnp.repeat(jnp.eye(_H, dtype=f32), _F, axis=1),
         jnp.zeros((12, _H * _F), f32)], axis=0)
    b1r = b1.reshape(1, -1)
    b2b = (b2 + bias).reshape(1, -1)

    y = pl.pallas_call(
        _mlp_body,
        grid=(158,),
        in_specs=[
            pl.BlockSpec((256, _ACC_W), lambda i: (i, 0)),
            pl.BlockSpec((16, 128), lambda i: (0, 0)),
            pl.BlockSpec((128, 256), lambda i: (0, 0)),
            pl.BlockSpec((1, 256), lambda i: (0, 0)),
            pl.BlockSpec((256, 128), lambda i: (0, 0)),
            pl.BlockSpec((1, 128), lambda i: (0, 0)),
        ],
        out_specs=pl.BlockSpec((256, 128), lambda i: (i, 0)),
        out_shape=jax.ShapeDtypeStruct((_BV * _NP, 128), f32),
    )(agg, r16, w1p, b1r, W2, b2b)

    y4 = y.reshape(_BV, _NP, 128)[:, :_N, :]
    return y4.reshape(_B, _V, _N, _H * _F)
